# baseline (speedup 1.0000x reference)
import functools

import jax
import jax.numpy as jnp
from jax import lax
from jax.experimental import pallas as pl
from jax.experimental.pallas import tpu as pltpu
from jax.experimental.pallas import tpu_sc as plsc

LANES = 16


@functools.lru_cache(maxsize=None)
def _build(B, H, V, D, NC, NS):
    NW = NC * NS
    assert D == NW
    CB = 1024
    n_chunks = B // CB
    n_groups = CB // LANES

    mesh = plsc.VectorSubcoreMesh(core_axis_name="c", subcore_axis_name="s")

    @functools.partial(
        pl.kernel,
        mesh=mesh,
        out_type=jax.ShapeDtypeStruct((D * B,), jnp.float32),
        scratch_types=[
            pltpu.VMEM((V,), jnp.float32),
            pltpu.VMEM((H, CB), jnp.int32),
            pltpu.VMEM((CB,), jnp.float32),
            pltpu.SemaphoreType.DMA,
        ],
        compiler_params=pltpu.CompilerParams(
            use_tc_tiling_on_sc=False, needs_layout_passes=False
        ),
    )
    def emb(idx_hbm, wt_hbm, out_hbm, row_v, iv, out_v, sem):
        d = lax.axis_index("s") * NC + lax.axis_index("c")
        r0 = pl.multiple_of(d * V, 8)
        pltpu.sync_copy(wt_hbm.at[pl.ds(r0, V)], row_v)
        o0 = pl.multiple_of(d * B, 8)

        def chunk(c, _):
            b0 = c * CB
            copies = [
                pltpu.async_copy(
                    idx_hbm.at[pl.ds(h * B + b0, CB)], iv.at[h], sem
                )
                for h in range(H)
            ]
            for cp in copies:
                cp.wait()

            @plsc.parallel_loop(0, n_groups)
            def group(g):
                g0 = g * LANES
                acc = plsc.load_gather(row_v, [iv[0, pl.ds(g0, LANES)]])
                for h in range(1, H):
                    acc = acc + plsc.load_gather(row_v, [iv[h, pl.ds(g0, LANES)]])
                out_v[pl.ds(g0, LANES)] = acc

            pltpu.sync_copy(out_v, out_hbm.at[pl.ds(o0 + b0, CB)])
            return 0

        lax.fori_loop(0, n_chunks, chunk, 0)

    return emb


def kernel(input, weight):
    B, H = input.shape
    V, D = weight.shape
    info = plsc.get_sparse_core_info()
    emb = _build(B, H, V, D, info.num_cores, info.num_subcores)
    idx_flat = input.T.reshape(H * B)
    wt_flat = weight.T.reshape(D * V)
    out_t = emb(idx_flat, wt_flat)
    return out_t.reshape(D, B).T
